# baseline (speedup 1.0000x reference)
import functools

import jax
import jax.numpy as jnp
from jax import lax
from jax.experimental import pallas as pl
from jax.experimental.pallas import tpu as pltpu
from jax.experimental.pallas import tpu_sc as plsc

N = 10000
E = 160000
H = 256
G = 128
OUT = 2
NF = 56

NC = 2
NS = 16
CH = 128
NCHT = 80
EPT = NCHT * CH
EP = NS * EPT
NCHW = 40
SL = 32
ACH = 64
ANCH = EPT // ACH
NPAD = 10112
ZR = NPAD // NS
ZRL = N - (NS - 1) * ZR
HH = H // 2
RB = 400
NB = N // RB

@functools.cache
def _sc_deg_kernel():
    mesh = plsc.VectorSubcoreMesh(
        core_axis_name="c", subcore_axis_name="s",
        num_cores=NC, num_subcores=NS)
    return pl.kernel(
        _sc_deg_body,
        out_type=jax.ShapeDtypeStruct((NC * N, HH), jnp.float32),
        mesh=mesh,
        scratch_types=[
            pltpu.VMEM((NCHW, CH), jnp.int32),
            pltpu.VMEM((CH, HH), jnp.float32),
            pltpu.VMEM_SHARED((NPAD, HH), jnp.float32),
        ],
    )


def _sc_deg(dstw, onesr, zn):
    return _sc_deg_kernel()(dstw, onesr, zn)


def _sc_deg_body(dstw_hbm, ones_hbm, zeros_hbm, out_hbm, didx, ones_v, acc):
    c = lax.axis_index("c")
    s = lax.axis_index("s")
    wid = s * NC + c
    pltpu.sync_copy(zeros_hbm, acc.at[pl.ds(s * ZR, ZR)])
    pltpu.sync_copy(dstw_hbm.at[wid], didx)
    pltpu.sync_copy(ones_hbm, ones_v)
    plsc.subcore_barrier()

    def step(j, carry):
        pltpu.sync_copy(ones_v, acc.at[didx.at[j]], add=True)
        return carry

    lax.fori_loop(0, NCHW, step, 0)
    plsc.subcore_barrier()
    _copy_out(acc, out_hbm, c, s)


def _copy_out(acc, out_hbm, c, s):
    @pl.when(s < NS - 1)
    def _full():
        pltpu.sync_copy(acc.at[pl.ds(s * ZR, ZR)],
                        out_hbm.at[pl.ds(c * N + s * ZR, ZR)])

    @pl.when(s == NS - 1)
    def _last():
        pltpu.sync_copy(acc.at[pl.ds((NS - 1) * ZR, ZRL)],
                        out_hbm.at[pl.ds(c * N + (NS - 1) * ZR, ZRL)])


@functools.cache
def _sc_agg_kernel():
    mesh = plsc.VectorSubcoreMesh(
        core_axis_name="c", subcore_axis_name="s",
        num_cores=NC, num_subcores=NS)
    return pl.kernel(
        _sc_agg_body,
        out_type=jax.ShapeDtypeStruct((NC * N, HH), jnp.float32),
        mesh=mesh,
        scratch_types=[
            pltpu.VMEM((SL, ACH), jnp.int32),
            pltpu.VMEM((SL, ACH), jnp.int32),
            pltpu.VMEM((ACH, HH), jnp.float32),
            pltpu.VMEM((ACH, HH), jnp.float32),
            pltpu.VMEM((ACH, HH), jnp.float32),
            pltpu.VMEM((ACH, HH), jnp.float32),
            pltpu.VMEM_SHARED((NPAD, HH), jnp.float32),
            pltpu.SemaphoreType.DMA,
            pltpu.SemaphoreType.DMA,
            pltpu.SemaphoreType.DMA,
            pltpu.SemaphoreType.DMA,
        ],
    )


def _sc_agg(table, srcb, dstr, zn):
    return _sc_agg_kernel()(table, srcb, dstr, zn)


def _sc_agg_body(table_hbm, src_hbm, dst_hbm, zeros_hbm, out_hbm,
                 sidx, didx, buf0, buf1, buf2, buf3, acc,
                 sem0, sem1, sem2, sem3):
    c = lax.axis_index("c")
    s = lax.axis_index("s")
    bufs = (buf0, buf1, buf2, buf3)
    sems = (sem0, sem1, sem2, sem3)
    pltpu.sync_copy(zeros_hbm, acc.at[pl.ds(s * ZR, ZR)])
    plsc.subcore_barrier()

    def slab(k, carry):
        pltpu.sync_copy(src_hbm.at[c, s, pl.ds(k * SL, SL)], sidx)
        pltpu.sync_copy(dst_hbm.at[s, pl.ds(k * SL, SL)], didx)
        for p in range(3):
            pltpu.async_copy(table_hbm.at[sidx.at[p]], bufs[p], sems[p])

        def step(j, cc):
            for p in range(4):
                @pl.when(j % 4 == p)
                def _(p=p):
                    @pl.when(j + 3 < SL)
                    def _issue():
                        q = (p + 3) % 4
                        pltpu.async_copy(
                            table_hbm.at[sidx.at[j + 3]], bufs[q], sems[q])

                    pltpu.make_async_copy(
                        table_hbm.at[sidx.at[j]], bufs[p], sems[p]).wait()
                    pltpu.sync_copy(bufs[p], acc.at[didx.at[j]], add=True)

            return cc

        lax.fori_loop(0, SL, step, 0)
        return carry

    lax.fori_loop(0, ANCH // SL, slab, 0)
    plsc.subcore_barrier()
    _copy_out(acc, out_hbm, c, s)


_BN_RS = 1.0 / (1.0 + 1e-5) ** 0.5


def _dis_block(d0, d1):
    deg = d0[:, :1] + d1[:, :1] + 1.0
    return lax.rsqrt(deg)


def _k1_body(x_ref, e0_ref, e1_ref, w_ref, d0_ref, d1_ref, o_ref):
    xb = x_ref[...].astype(jnp.float32)
    dmat = e1_ref[...] - e0_ref[...]
    base = jnp.sum(e0_ref[...], axis=0, keepdims=True)
    h0 = jnp.dot(xb, dmat, preferred_element_type=jnp.float32) + base
    dis = _dis_block(d0_ref[...], d1_ref[...])
    o_ref[...] = dis * jnp.dot(h0, w_ref[...], preferred_element_type=jnp.float32)


def _mid_body(a0_ref, a1_ref, h0_ref, h1_ref, d0_ref, d1_ref,
              b_ref, g_ref, be_ref, w_ref, o_ref):
    dis = _dis_block(d0_ref[...], d1_ref[...])
    pre = jnp.concatenate(
        [a0_ref[...] + h0_ref[...], a1_ref[...] + h1_ref[...]], axis=1)
    hb = dis * pre + b_ref[...]
    hb = jnp.maximum(hb * (g_ref[...] * _BN_RS) + be_ref[...], 0.0)
    o_ref[...] = dis * jnp.dot(hb, w_ref[...], preferred_element_type=jnp.float32)


def _k3_tail_body(a0_ref, a1_ref, h0_ref, h1_ref, d0_ref, d1_ref,
                  b_ref, bat_ref, wl_ref, bl_ref, o_ref, psum, csum):
    i = pl.program_id(0)

    @pl.when(i == 0)
    def _init():
        psum[...] = jnp.zeros_like(psum)
        csum[...] = jnp.zeros_like(csum)

    dis = _dis_block(d0_ref[...], d1_ref[...])
    pre = jnp.concatenate(
        [a0_ref[...] + h0_ref[...], a1_ref[...] + h1_ref[...]], axis=1)
    h3 = dis * pre + b_ref[...]
    bat = bat_ref[0]
    gid = lax.broadcasted_iota(jnp.int32, (G, RB), 0)
    p = (gid == bat).astype(jnp.float32)
    psum[...] += jnp.dot(p, h3, preferred_element_type=jnp.float32)
    csum[...] += jnp.sum(p, axis=1)[:, None]

    @pl.when(i == NB - 1)
    def _fin():
        pooled = psum[...] / jnp.maximum(csum[...], 1.0)
        o_ref[...] = (jnp.dot(pooled, wl_ref[...],
                              preferred_element_type=jnp.float32) + bl_ref[...])


def _deg_specs():
    return [
        pl.BlockSpec((RB, HH), lambda i, h: (i, 0)),
        pl.BlockSpec((RB, HH), lambda i, h: (NB + i, 0)),
    ]


def _tc_k1(x, emb0, emb1, w1, degp):
    return pl.pallas_call(
        _k1_body,
        grid=(NB, 2),
        in_specs=[
            pl.BlockSpec((RB, NF), lambda i, h: (i, 0)),
            pl.BlockSpec((NF, H), lambda i, h: (0, 0)),
            pl.BlockSpec((NF, H), lambda i, h: (0, 0)),
            pl.BlockSpec((H, HH), lambda i, h: (0, h)),
            *_deg_specs(),
        ],
        out_specs=pl.BlockSpec((RB, HH), lambda i, h: (h * NB + i, 0)),
        out_shape=jax.ShapeDtypeStruct((2 * N, HH), jnp.float32),
    )(x, emb0, emb1, w1, degp, degp)


def _tc_mid(aggp, hws, degp, b, g, be, w):
    return pl.pallas_call(
        _mid_body,
        grid=(NB, 2),
        in_specs=[
            pl.BlockSpec((RB, HH), lambda i, h: (i, 0)),
            pl.BlockSpec((RB, HH), lambda i, h: (NB + i, 0)),
            pl.BlockSpec((RB, HH), lambda i, h: (i, 0)),
            pl.BlockSpec((RB, HH), lambda i, h: (NB + i, 0)),
            *_deg_specs(),
            pl.BlockSpec((1, H), lambda i, h: (0, 0)),
            pl.BlockSpec((1, H), lambda i, h: (0, 0)),
            pl.BlockSpec((1, H), lambda i, h: (0, 0)),
            pl.BlockSpec((H, HH), lambda i, h: (0, h)),
        ],
        out_specs=pl.BlockSpec((RB, HH), lambda i, h: (h * NB + i, 0)),
        out_shape=jax.ShapeDtypeStruct((2 * N, HH), jnp.float32),
    )(aggp, aggp, hws, hws, degp, degp, b, g, be, w)


def _tc_k3_tail(aggp, hws, degp, b3, batr, wl, bl):
    return pl.pallas_call(
        _k3_tail_body,
        grid=(NB,),
        in_specs=[
            pl.BlockSpec((RB, HH), lambda i: (i, 0)),
            pl.BlockSpec((RB, HH), lambda i: (NB + i, 0)),
            pl.BlockSpec((RB, HH), lambda i: (i, 0)),
            pl.BlockSpec((RB, HH), lambda i: (NB + i, 0)),
            pl.BlockSpec((RB, HH), lambda i: (i, 0)),
            pl.BlockSpec((RB, HH), lambda i: (NB + i, 0)),
            pl.BlockSpec((1, H), lambda i: (0, 0)),
            pl.BlockSpec((1, 1, RB), lambda i: (i, 0, 0)),
            pl.BlockSpec((H, OUT), lambda i: (0, 0)),
            pl.BlockSpec((1, OUT), lambda i: (0, 0)),
        ],
        out_specs=pl.BlockSpec((G, OUT), lambda i: (0, 0)),
        out_shape=jax.ShapeDtypeStruct((G, OUT), jnp.float32),
        scratch_shapes=[
            pltpu.VMEM((G, H), jnp.float32),
            pltpu.VMEM((G, 1), jnp.float32),
        ],
    )(aggp, aggp, hws, hws, degp, degp, b3, batr, wl, bl)


def kernel(x, edge_index, batch, emb_tables, W1, b1, g1, be1,
           W2, b2, g2, be2, W3, b3, Wl, bl):
    src = edge_index[0]
    dst = edge_index[1]
    pad = EP - E
    srcp = jnp.concatenate([src, jnp.zeros((pad,), jnp.int32)])
    dstp = jnp.concatenate([dst, jnp.full((pad,), N, jnp.int32)])
    srcr = srcp.reshape(NS, ANCH, ACH)
    srcb = jnp.stack([srcr, srcr + N])
    dstr = dstp.reshape(NS, ANCH, ACH)
    dstw = dstp.reshape(NC * NS, NCHW, CH)

    zn = jnp.zeros((ZR, HH), jnp.float32)
    onesr = jnp.broadcast_to(
        (jnp.arange(HH) == 0).astype(jnp.float32)[None, :], (CH, HH))

    emb0 = emb_tables[:, 0, :]
    emb1 = emb_tables[:, 1, :]
    batr = batch.reshape(NB, 1, RB)
    b1r, g1r, be1r = b1[None], g1[None], be1[None]
    b2r, g2r, be2r = b2[None], g2[None], be2[None]
    b3r, blr = b3[None], bl[None]

    degp = _sc_deg(dstw, onesr, zn)
    hws1 = _tc_k1(x, emb0, emb1, W1, degp)
    agg1 = _sc_agg(hws1, srcb, dstr, zn)
    hws2 = _tc_mid(agg1, hws1, degp, b1r, g1r, be1r, W2)
    agg2 = _sc_agg(hws2, srcb, dstr, zn)
    hws3 = _tc_mid(agg2, hws2, degp, b2r, g2r, be2r, W3)
    agg3 = _sc_agg(hws3, srcb, dstr, zn)
    return _tc_k3_tail(agg3, hws3, degp, b3r, batr, Wl, blr)


# speedup vs baseline: 1.0070x; 1.0070x over previous
import functools

import jax
import jax.numpy as jnp
from jax import lax
from jax.experimental import pallas as pl
from jax.experimental.pallas import tpu as pltpu
from jax.experimental.pallas import tpu_sc as plsc

N = 10000
E = 160000
H = 256
G = 128
OUT = 2
NF = 56

NC = 2
NS = 16
CH = 128
NCHT = 80
EPT = NCHT * CH
EP = NS * EPT
NCHW = 40
SL = 32
ACH = 64
ANCH = EPT // ACH
NPAD = 10112
ZR = NPAD // NS
ZRL = N - (NS - 1) * ZR
HH = H // 2
RB = 400
NB = N // RB

@functools.cache
def _sc_deg_kernel():
    mesh = plsc.VectorSubcoreMesh(
        core_axis_name="c", subcore_axis_name="s",
        num_cores=NC, num_subcores=NS)
    return pl.kernel(
        _sc_deg_body,
        out_type=jax.ShapeDtypeStruct((NC * N, HH), jnp.float32),
        mesh=mesh,
        scratch_types=[
            pltpu.VMEM((NCHW, CH), jnp.int32),
            pltpu.VMEM((CH, HH), jnp.float32),
            pltpu.VMEM_SHARED((NPAD, HH), jnp.float32),
        ],
    )


def _sc_deg(dstw, onesr, zn):
    return _sc_deg_kernel()(dstw, onesr, zn)


def _sc_deg_body(dstw_hbm, ones_hbm, zeros_hbm, out_hbm, didx, ones_v, acc):
    c = lax.axis_index("c")
    s = lax.axis_index("s")
    wid = s * NC + c
    pltpu.sync_copy(zeros_hbm, acc.at[pl.ds(s * ZR, ZR)])
    pltpu.sync_copy(dstw_hbm.at[wid], didx)
    pltpu.sync_copy(ones_hbm, ones_v)
    plsc.subcore_barrier()

    def step(j, carry):
        pltpu.sync_copy(ones_v, acc.at[didx.at[j]], add=True)
        return carry

    lax.fori_loop(0, NCHW, step, 0)
    plsc.subcore_barrier()
    _copy_out(acc, out_hbm, c, s)


def _copy_out(acc, out_hbm, c, s):
    @pl.when(s < NS - 1)
    def _full():
        pltpu.sync_copy(acc.at[pl.ds(s * ZR, ZR)],
                        out_hbm.at[pl.ds(c * N + s * ZR, ZR)])

    @pl.when(s == NS - 1)
    def _last():
        pltpu.sync_copy(acc.at[pl.ds((NS - 1) * ZR, ZRL)],
                        out_hbm.at[pl.ds(c * N + (NS - 1) * ZR, ZRL)])


@functools.cache
def _sc_agg_kernel():
    mesh = plsc.VectorSubcoreMesh(
        core_axis_name="c", subcore_axis_name="s",
        num_cores=NC, num_subcores=NS)
    return pl.kernel(
        _sc_agg_body,
        out_type=jax.ShapeDtypeStruct((NC * N, HH), jnp.float32),
        mesh=mesh,
        scratch_types=[
            pltpu.VMEM((SL, ACH), jnp.int32),
            pltpu.VMEM((SL, ACH), jnp.int32),
            pltpu.VMEM((ACH, HH), jnp.float32),
            pltpu.VMEM((ACH, HH), jnp.float32),
            pltpu.VMEM((ACH, HH), jnp.float32),
            pltpu.VMEM((ACH, HH), jnp.float32),
            pltpu.VMEM((ACH, HH), jnp.float32),
            pltpu.VMEM_SHARED((NPAD, HH), jnp.float32),
            pltpu.SemaphoreType.DMA,
            pltpu.SemaphoreType.DMA,
            pltpu.SemaphoreType.DMA,
            pltpu.SemaphoreType.DMA,
            pltpu.SemaphoreType.DMA,
        ],
    )


def _sc_agg(table, srcb, dstr, zn):
    return _sc_agg_kernel()(table, srcb, dstr, zn)


def _sc_agg_body(table_hbm, src_hbm, dst_hbm, zeros_hbm, out_hbm,
                 sidx, didx, buf0, buf1, buf2, buf3, buf4, acc,
                 sem0, sem1, sem2, sem3, sem4):
    c = lax.axis_index("c")
    s = lax.axis_index("s")
    bufs = (buf0, buf1, buf2, buf3, buf4)
    sems = (sem0, sem1, sem2, sem3, sem4)
    pltpu.sync_copy(zeros_hbm, acc.at[pl.ds(s * ZR, ZR)])
    plsc.subcore_barrier()

    def slab(k, carry):
        pltpu.sync_copy(src_hbm.at[c, s, pl.ds(k * SL, SL)], sidx)
        pltpu.sync_copy(dst_hbm.at[s, pl.ds(k * SL, SL)], didx)
        for p in range(4):
            pltpu.async_copy(table_hbm.at[sidx.at[p]], bufs[p], sems[p])

        def step(j, cc):
            for p in range(5):
                @pl.when(j % 5 == p)
                def _(p=p):
                    @pl.when(j + 4 < SL)
                    def _issue():
                        q = (p + 4) % 5
                        pltpu.async_copy(
                            table_hbm.at[sidx.at[j + 4]], bufs[q], sems[q])

                    pltpu.make_async_copy(
                        table_hbm.at[sidx.at[j]], bufs[p], sems[p]).wait()
                    pltpu.sync_copy(bufs[p], acc.at[didx.at[j]], add=True)

            return cc

        lax.fori_loop(0, SL, step, 0)
        return carry

    lax.fori_loop(0, ANCH // SL, slab, 0)
    plsc.subcore_barrier()
    _copy_out(acc, out_hbm, c, s)


_BN_RS = 1.0 / (1.0 + 1e-5) ** 0.5


def _dis_block(d0, d1):
    deg = d0[:, :1] + d1[:, :1] + 1.0
    return lax.rsqrt(deg)


def _k1_body(x_ref, e0_ref, e1_ref, w_ref, d0_ref, d1_ref, o_ref):
    xb = x_ref[...].astype(jnp.float32)
    dmat = e1_ref[...] - e0_ref[...]
    base = jnp.sum(e0_ref[...], axis=0, keepdims=True)
    h0 = jnp.dot(xb, dmat, preferred_element_type=jnp.float32) + base
    dis = _dis_block(d0_ref[...], d1_ref[...])
    o_ref[...] = dis * jnp.dot(h0, w_ref[...], preferred_element_type=jnp.float32)


def _mid_body(a0_ref, a1_ref, h0_ref, h1_ref, d0_ref, d1_ref,
              b_ref, g_ref, be_ref, w_ref, o_ref):
    dis = _dis_block(d0_ref[...], d1_ref[...])
    pre = jnp.concatenate(
        [a0_ref[...] + h0_ref[...], a1_ref[...] + h1_ref[...]], axis=1)
    hb = dis * pre + b_ref[...]
    hb = jnp.maximum(hb * (g_ref[...] * _BN_RS) + be_ref[...], 0.0)
    o_ref[...] = dis * jnp.dot(hb, w_ref[...], preferred_element_type=jnp.float32)


def _k3_tail_body(a0_ref, a1_ref, h0_ref, h1_ref, d0_ref, d1_ref,
                  b_ref, bat_ref, wl_ref, bl_ref, o_ref, psum, csum):
    i = pl.program_id(0)

    @pl.when(i == 0)
    def _init():
        psum[...] = jnp.zeros_like(psum)
        csum[...] = jnp.zeros_like(csum)

    dis = _dis_block(d0_ref[...], d1_ref[...])
    pre = jnp.concatenate(
        [a0_ref[...] + h0_ref[...], a1_ref[...] + h1_ref[...]], axis=1)
    h3 = dis * pre + b_ref[...]
    bat = bat_ref[0]
    gid = lax.broadcasted_iota(jnp.int32, (G, RB), 0)
    p = (gid == bat).astype(jnp.float32)
    psum[...] += jnp.dot(p, h3, preferred_element_type=jnp.float32)
    csum[...] += jnp.sum(p, axis=1)[:, None]

    @pl.when(i == NB - 1)
    def _fin():
        pooled = psum[...] / jnp.maximum(csum[...], 1.0)
        o_ref[...] = (jnp.dot(pooled, wl_ref[...],
                              preferred_element_type=jnp.float32) + bl_ref[...])


def _deg_specs():
    return [
        pl.BlockSpec((RB, HH), lambda h, i: (i, 0)),
        pl.BlockSpec((RB, HH), lambda h, i: (NB + i, 0)),
    ]


def _tc_k1(x, emb0, emb1, w1, degp):
    return pl.pallas_call(
        _k1_body,
        grid=(2, NB),
        in_specs=[
            pl.BlockSpec((RB, NF), lambda h, i: (i, 0)),
            pl.BlockSpec((NF, H), lambda h, i: (0, 0)),
            pl.BlockSpec((NF, H), lambda h, i: (0, 0)),
            pl.BlockSpec((H, HH), lambda h, i: (0, h)),
            *_deg_specs(),
        ],
        out_specs=pl.BlockSpec((RB, HH), lambda h, i: (h * NB + i, 0)),
        out_shape=jax.ShapeDtypeStruct((2 * N, HH), jnp.float32),
    )(x, emb0, emb1, w1, degp, degp)


def _tc_mid(aggp, hws, degp, b, g, be, w):
    return pl.pallas_call(
        _mid_body,
        grid=(2, NB),
        in_specs=[
            pl.BlockSpec((RB, HH), lambda h, i: (i, 0)),
            pl.BlockSpec((RB, HH), lambda h, i: (NB + i, 0)),
            pl.BlockSpec((RB, HH), lambda h, i: (i, 0)),
            pl.BlockSpec((RB, HH), lambda h, i: (NB + i, 0)),
            *_deg_specs(),
            pl.BlockSpec((1, H), lambda h, i: (0, 0)),
            pl.BlockSpec((1, H), lambda h, i: (0, 0)),
            pl.BlockSpec((1, H), lambda h, i: (0, 0)),
            pl.BlockSpec((H, HH), lambda h, i: (0, h)),
        ],
        out_specs=pl.BlockSpec((RB, HH), lambda h, i: (h * NB + i, 0)),
        out_shape=jax.ShapeDtypeStruct((2 * N, HH), jnp.float32),
    )(aggp, aggp, hws, hws, degp, degp, b, g, be, w)


def _tc_k3_tail(aggp, hws, degp, b3, batr, wl, bl):
    return pl.pallas_call(
        _k3_tail_body,
        grid=(NB,),
        in_specs=[
            pl.BlockSpec((RB, HH), lambda i: (i, 0)),
            pl.BlockSpec((RB, HH), lambda i: (NB + i, 0)),
            pl.BlockSpec((RB, HH), lambda i: (i, 0)),
            pl.BlockSpec((RB, HH), lambda i: (NB + i, 0)),
            pl.BlockSpec((RB, HH), lambda i: (i, 0)),
            pl.BlockSpec((RB, HH), lambda i: (NB + i, 0)),
            pl.BlockSpec((1, H), lambda i: (0, 0)),
            pl.BlockSpec((1, 1, RB), lambda i: (i, 0, 0)),
            pl.BlockSpec((H, OUT), lambda i: (0, 0)),
            pl.BlockSpec((1, OUT), lambda i: (0, 0)),
        ],
        out_specs=pl.BlockSpec((G, OUT), lambda i: (0, 0)),
        out_shape=jax.ShapeDtypeStruct((G, OUT), jnp.float32),
        scratch_shapes=[
            pltpu.VMEM((G, H), jnp.float32),
            pltpu.VMEM((G, 1), jnp.float32),
        ],
    )(aggp, aggp, hws, hws, degp, degp, b3, batr, wl, bl)


def kernel(x, edge_index, batch, emb_tables, W1, b1, g1, be1,
           W2, b2, g2, be2, W3, b3, Wl, bl):
    src = edge_index[0]
    dst = edge_index[1]
    pad = EP - E
    srcp = jnp.concatenate([src, jnp.zeros((pad,), jnp.int32)])
    dstp = jnp.concatenate([dst, jnp.full((pad,), N, jnp.int32)])
    srcr = srcp.reshape(NS, ANCH, ACH)
    srcb = jnp.stack([srcr, srcr + N])
    dstr = dstp.reshape(NS, ANCH, ACH)
    dstw = dstp.reshape(NC * NS, NCHW, CH)

    zn = jnp.zeros((ZR, HH), jnp.float32)
    onesr = jnp.broadcast_to(
        (jnp.arange(HH) == 0).astype(jnp.float32)[None, :], (CH, HH))

    emb0 = emb_tables[:, 0, :]
    emb1 = emb_tables[:, 1, :]
    batr = batch.reshape(NB, 1, RB)
    b1r, g1r, be1r = b1[None], g1[None], be1[None]
    b2r, g2r, be2r = b2[None], g2[None], be2[None]
    b3r, blr = b3[None], bl[None]

    degp = _sc_deg(dstw, onesr, zn)
    hws1 = _tc_k1(x, emb0, emb1, W1, degp)
    agg1 = _sc_agg(hws1, srcb, dstr, zn)
    hws2 = _tc_mid(agg1, hws1, degp, b1r, g1r, be1r, W2)
    agg2 = _sc_agg(hws2, srcb, dstr, zn)
    hws3 = _tc_mid(agg2, hws2, degp, b2r, g2r, be2r, W3)
    agg3 = _sc_agg(hws3, srcb, dstr, zn)
    return _tc_k3_tail(agg3, hws3, degp, b3r, batr, Wl, blr)
